# baseline (speedup 1.0000x reference)
import functools

import jax
import jax.numpy as jnp
from jax.experimental import pallas as pl
from jax.experimental.pallas import tpu as pltpu

_GAM1 = 0.01
_GAM2 = 0.01


def _pge_tc_kernel(q_ref, qa_ref, p_ref, lab_ref, out_ref, *,
                   n_classes, n_total, np_, c_pad):
    i = pl.program_id(0)
    q = q_ref[...]
    qa = qa_ref[...]
    bm = q.shape[0]
    q2 = jnp.sum(q * q, axis=1, keepdims=True)

    mn = jnp.full((bm, c_pad), jnp.inf, jnp.float32)
    mx = jnp.full((bm, c_pad), -jnp.inf, jnp.float32)
    for k in range(np_):
        pk = p_ref[k]
        t = jax.lax.dot_general(
            qa, pk, (((1,), (1,)), ((), ())),
            preferred_element_type=jnp.float32)
        mn = jnp.minimum(mn, t)
        mx = jnp.maximum(mx, t)

    mind = jnp.sqrt(jnp.maximum(mn + q2, 1e-12))
    maxd = jnp.sqrt(jnp.maximum(mx + q2, 1e-12))

    cls = jax.lax.broadcasted_iota(jnp.int32, (bm, c_pad), 1)
    valid = cls < n_classes
    own = lab_ref[...] == cls

    s_all_min = jnp.sum(jnp.where(valid, mind, 0.0))
    s_own_min = jnp.sum(jnp.where(own, mind, 0.0))
    s_own_max = jnp.sum(jnp.where(own, maxd, 0.0))

    part = (_GAM1 / n_total) * s_own_max \
        - (_GAM2 / (n_total * (n_classes - 1))) * (s_all_min - s_own_min)

    @pl.when(i == 0)
    def _init():
        out_ref[0, 0] = jnp.float32(0.0)

    out_ref[0, 0] += part


def kernel(queries, pivots, labels):
    B, d = queries.shape
    C, Np, _ = pivots.shape
    c_pad = 512
    bm = 256

    p = jnp.transpose(pivots, (1, 0, 2))
    p = jnp.pad(p, ((0, 0), (0, c_pad - C), (0, 0)))
    p2 = jnp.sum(p * p, axis=-1)
    p2_hi = p2.astype(jnp.bfloat16)
    p2_lo = (p2 - p2_hi.astype(jnp.float32)).astype(jnp.bfloat16)
    p_aug = jnp.concatenate(
        [(-2.0 * p).astype(jnp.bfloat16),
         p2_hi[:, :, None], p2_lo[:, :, None]], axis=2)

    q_aug = jnp.concatenate(
        [queries.astype(jnp.bfloat16),
         jnp.ones((B, 2), jnp.bfloat16)], axis=1)
    lab = labels.astype(jnp.int32).reshape(B, 1)

    grid = (B // bm,)
    out = pl.pallas_call(
        functools.partial(_pge_tc_kernel, n_classes=C, n_total=B, np_=Np,
                          c_pad=c_pad),
        grid=grid,
        in_specs=[
            pl.BlockSpec((bm, d), lambda i: (i, 0)),
            pl.BlockSpec((bm, d + 2), lambda i: (i, 0)),
            pl.BlockSpec((Np, c_pad, d + 2), lambda i: (0, 0, 0)),
            pl.BlockSpec((bm, 1), lambda i: (i, 0)),
        ],
        out_specs=pl.BlockSpec(memory_space=pltpu.SMEM),
        out_shape=jax.ShapeDtypeStruct((1, 1), jnp.float32),
        compiler_params=pltpu.CompilerParams(
            dimension_semantics=("arbitrary",)),
    )(queries, q_aug, p_aug, lab)
    return out[0, 0]


# speedup vs baseline: 1.4969x; 1.4969x over previous
import functools

import jax
import jax.numpy as jnp
from jax.experimental import pallas as pl
from jax.experimental.pallas import tpu as pltpu

_GAM1 = 0.01
_GAM2 = 0.01


def _pge_tc_kernel(q_ref, qa_ref, p_ref, p2_ref, lab_ref, out_ref, *,
                   n_classes, n_total, np_, c_pad):
    i = pl.program_id(0)
    q = q_ref[...]
    qa = qa_ref[...]
    bm = q.shape[0]
    q2 = jnp.sum(q * q, axis=1, keepdims=True)

    mn = jnp.full((bm, c_pad), jnp.inf, jnp.float32)
    mx = jnp.full((bm, c_pad), -jnp.inf, jnp.float32)
    for k in range(np_):
        pk = p_ref[k]
        qp = jax.lax.dot_general(
            qa, pk, (((1,), (1,)), ((), ())),
            preferred_element_type=jnp.float32)
        t = p2_ref[k] + qp
        mn = jnp.minimum(mn, t)
        mx = jnp.maximum(mx, t)

    mind = jnp.sqrt(jnp.maximum(mn + q2, 1e-12))
    maxd = jnp.sqrt(jnp.maximum(mx + q2, 1e-12))

    cls = jax.lax.broadcasted_iota(jnp.int32, (bm, c_pad), 1)
    valid = cls < n_classes
    own = lab_ref[...] == cls

    s_all_min = jnp.sum(jnp.where(valid, mind, 0.0))
    s_own_min = jnp.sum(jnp.where(own, mind, 0.0))
    s_own_max = jnp.sum(jnp.where(own, maxd, 0.0))

    part = (_GAM1 / n_total) * s_own_max \
        - (_GAM2 / (n_total * (n_classes - 1))) * (s_all_min - s_own_min)

    @pl.when(i == 0)
    def _init():
        out_ref[0, 0] = jnp.float32(0.0)

    out_ref[0, 0] += part


def kernel(queries, pivots, labels):
    B, d = queries.shape
    C, Np, _ = pivots.shape
    c_pad = 512
    bm = 256

    p = jnp.transpose(pivots, (1, 0, 2))
    p = jnp.pad(p, ((0, 0), (0, c_pad - C), (0, 0)))
    p2 = jnp.sum(p * p, axis=-1)[:, None, :]
    p_aug = (-2.0 * p).astype(jnp.bfloat16)
    q_aug = queries.astype(jnp.bfloat16)
    lab = labels.astype(jnp.int32).reshape(B, 1)

    grid = (B // bm,)
    out = pl.pallas_call(
        functools.partial(_pge_tc_kernel, n_classes=C, n_total=B, np_=Np,
                          c_pad=c_pad),
        grid=grid,
        in_specs=[
            pl.BlockSpec((bm, d), lambda i: (i, 0)),
            pl.BlockSpec((bm, d), lambda i: (i, 0)),
            pl.BlockSpec((Np, c_pad, d), lambda i: (0, 0, 0)),
            pl.BlockSpec((Np, 1, c_pad), lambda i: (0, 0, 0)),
            pl.BlockSpec((bm, 1), lambda i: (i, 0)),
        ],
        out_specs=pl.BlockSpec(memory_space=pltpu.SMEM),
        out_shape=jax.ShapeDtypeStruct((1, 1), jnp.float32),
        compiler_params=pltpu.CompilerParams(
            dimension_semantics=("arbitrary",)),
    )(queries, q_aug, p_aug, p2, lab)
    return out[0, 0]
